# baseline (speedup 1.0000x reference)
import functools

import jax
import jax.numpy as jnp
from jax import lax
from jax.experimental import pallas as pl
from jax.experimental.pallas import tpu as pltpu
from jax.experimental.pallas import tpu_sc as plsc

N = 5000
D = 128
E = 320000
NC = 2
NS = 16
NPAD = 5120
NPT = NPAD // NS
K = 128
NCHUNK = 2560
EPAD = NCHUNK * K
CPT = NCHUNK // NS
HCPT = CPT // 2
KR = 64
ECH = EPAD // KR
CPT2 = ECH // NS
QC = CPT2 // 4
SQ = 56
HQ = QC - SQ
PADROW = NPAD - 1
EPK = EPAD // NS

_mesh = plsc.VectorSubcoreMesh(
    core_axis_name="c", subcore_axis_name="s", num_cores=NC, num_subcores=NS)

_sc_params = pltpu.CompilerParams(use_tc_tiling_on_sc=False)

_f32 = jnp.float32
_i32 = jnp.int32


@functools.partial(
    pl.kernel,
    out_type=jax.ShapeDtypeStruct((6, NPAD), _f32),
    mesh=_mesh,
    compiler_params=_sc_params,
    scratch_types=[
        pltpu.VMEM((2, EPK), _i32),
        pltpu.VMEM((EPK,), _f32),
        pltpu.VMEM((NPT,), _f32),
        [pltpu.VMEM_SHARED((NPAD,), _f32) for _ in range(3)],
        [pltpu.SemaphoreType.DMA for _ in range(2)],
    ],
)
def _deg_kernel(e0, e1, e2, out, idx_v, ones_v, buf_v, hists, dsems):
    c = lax.axis_index("c")
    s = lax.axis_index("s")

    @pl.loop(0, EPK // 16)
    def _(q):
        ones_v[pl.ds(q * 16, 16)] = jnp.ones((16,), _f32)

    @pl.loop(0, NPT // 16)
    def _(q):
        buf_v[pl.ds(q * 16, 16)] = jnp.zeros((16,), _f32)

    for h in hists:
        pltpu.sync_copy(buf_v, h.at[pl.ds(s * NPT, NPT)])
    plsc.subcore_barrier()

    es = (e0, e1, e2)
    for li in range(3):
        b = li % 2
        if li >= 2:
            pltpu.make_async_copy(ones_v, hists[li - 2].at[idx_v.at[b]],
                                  dsems[b]).wait()
        pltpu.sync_copy(es[li].at[c, pl.ds(s * EPK, EPK)], idx_v.at[b])
        pltpu.async_copy(ones_v, hists[li].at[idx_v.at[b]], dsems[b],
                         add=True)
    for li in (1, 2):
        pltpu.make_async_copy(ones_v, hists[li].at[idx_v.at[li % 2]],
                              dsems[li % 2]).wait()

    plsc.subcore_barrier()
    for li in range(3):
        pltpu.sync_copy(hists[li].at[pl.ds(s * NPT, NPT)], buf_v)
        pltpu.sync_copy(buf_v, out.at[2 * li + c, pl.ds(s * NPT, NPT)])


@functools.partial(
    pl.kernel,
    out_type=(jax.ShapeDtypeStruct((NPAD, D), _f32),
              jax.ShapeDtypeStruct((NPAD, D), _f32)),
    mesh=_mesh,
    compiler_params=_sc_params,
    scratch_types=[
        pltpu.VMEM((QC, KR), _i32),
        pltpu.VMEM((QC, KR), _i32),
        pltpu.VMEM((4, KR, D), _f32),
        pltpu.VMEM_SHARED((NPAD, D), _f32),
        pltpu.VMEM_SHARED((NPAD, D), _f32),
        [pltpu.SemaphoreType.DMA for _ in range(4)],
        [pltpu.SemaphoreType.DMA for _ in range(4)],
    ],
)
def _edge_kernel(gu, gi, e, pu, pi, gv, sv, rb, tab, acc, gsems, ssems):
    c = lax.axis_index("c")
    s = lax.axis_index("s")

    @pl.when(c == 0)
    def _():
        pltpu.sync_copy(gi.at[pl.ds(s * NPT, NPT)],
                        tab.at[pl.ds(s * NPT, NPT)])

    @pl.when(c == 1)
    def _():
        pltpu.sync_copy(gu.at[pl.ds(s * NPT, NPT)],
                        tab.at[pl.ds(s * NPT, NPT)])

    @pl.loop(0, KR)
    def _(i):
        @pl.loop(0, D // 16)
        def _(q):
            rb[0, i, pl.ds(q * 16, 16)] = jnp.zeros((16,), _f32)

    for r in range(NPT // KR):
        pltpu.sync_copy(rb.at[0], acc.at[pl.ds(s * NPT + r * KR, KR)])
    plsc.subcore_barrier()

    for quarter in range(4):
        base = s * CPT2 + quarter * QC
        pltpu.sync_copy(e.at[1 - c, pl.ds(base, QC)], gv)
        pltpu.sync_copy(e.at[c, pl.ds(base, QC)], sv)
        for b in range(2):
            pltpu.async_copy(tab.at[gv.at[b]], rb.at[b], gsems[b])

        @pl.loop(0, QC, step=4)
        def _(j):
            for b in range(4):
                jj = j + b
                b2 = (b + 2) % 4
                pltpu.make_async_copy(tab.at[gv.at[jj]], rb.at[b],
                                      gsems[b]).wait()
                pltpu.async_copy(rb.at[b], acc.at[sv.at[jj]], ssems[b],
                                 add=True)

                @pl.when(jj >= 2)
                def _():
                    pltpu.make_async_copy(rb.at[b2], acc.at[sv.at[jj - 2]],
                                          ssems[b2]).wait()

                @pl.when(jj + 2 < QC)
                def _():
                    pltpu.async_copy(tab.at[gv.at[jj + 2]], rb.at[b2],
                                     gsems[b2])

        pltpu.make_async_copy(rb.at[2], acc.at[sv.at[QC - 2]],
                              ssems[2]).wait()
        pltpu.make_async_copy(rb.at[3], acc.at[sv.at[QC - 1]],
                              ssems[3]).wait()

    plsc.subcore_barrier()

    def readout(out):
        nr = NPT // KR
        for r in range(nr):
            b = r % 2
            if r >= 2:
                pltpu.make_async_copy(
                    rb.at[b], out.at[pl.ds(s * NPT + (r - 2) * KR, KR)],
                    gsems[b]).wait()
            pltpu.sync_copy(acc.at[pl.ds(s * NPT + r * KR, KR)], rb.at[b])
            pltpu.async_copy(rb.at[b], out.at[pl.ds(s * NPT + r * KR, KR)],
                             gsems[b])
        for r in (nr - 2, nr - 1):
            pltpu.make_async_copy(
                rb.at[r % 2], out.at[pl.ds(s * NPT + r * KR, KR)],
                gsems[r % 2]).wait()

    @pl.when(c == 0)
    def _():
        readout(pu)

    @pl.when(c == 1)
    def _():
        readout(pi)


_GRID = 8
_RB = NPAD // _GRID

_tab_spec = pl.BlockSpec((_RB, D), lambda i: (i, 0))
_scT_spec = pl.BlockSpec((_RB, 6), lambda i: (i, 0))


def _head(degT, hu, hi):
    def body(degT_ref, hu_ref, hi_ref, scT_ref, gu_ref, gi_ref):
        scT = lax.rsqrt(jnp.maximum(degT_ref[...], 1.0))
        scT_ref[...] = scT
        gu_ref[...] = hu_ref[...] * scT[:, 0:1]
        gi_ref[...] = hi_ref[...] * scT[:, 1:2]

    return pl.pallas_call(
        body,
        grid=(_GRID,),
        in_specs=[_scT_spec, _tab_spec, _tab_spec],
        out_specs=(_scT_spec, _tab_spec, _tab_spec),
        out_shape=(jax.ShapeDtypeStruct((NPAD, 6), _f32),
                   jax.ShapeDtypeStruct((NPAD, D), _f32),
                   jax.ShapeDtypeStruct((NPAD, D), _f32)),
    )(degT, hu, hi)


def _combine(pu, pi, scT, su, si, l):
    def body(pu_ref, pi_ref, scT_ref, su_ref, si_ref,
             nsu_ref, nsi_ref, gu_ref, gi_ref):
        sc = scT_ref[...]
        hu1 = sc[:, 2 * l:2 * l + 1] * pu_ref[...]
        hi1 = sc[:, 2 * l + 1:2 * l + 2] * pi_ref[...]
        nsu_ref[...] = su_ref[...] + hu1
        nsi_ref[...] = si_ref[...] + hi1
        gu_ref[...] = sc[:, 2 * l + 2:2 * l + 3] * hu1
        gi_ref[...] = sc[:, 2 * l + 3:2 * l + 4] * hi1

    return pl.pallas_call(
        body,
        grid=(_GRID,),
        in_specs=[_tab_spec, _tab_spec, _scT_spec, _tab_spec, _tab_spec],
        out_specs=(_tab_spec,) * 4,
        out_shape=(jax.ShapeDtypeStruct((NPAD, D), _f32),) * 4,
    )(pu, pi, scT, su, si)


def _final(pu, pi, scT, su, si):
    def body(pu_ref, pi_ref, scT_ref, su_ref, si_ref, ou_ref, oi_ref):
        sc = scT_ref[...]
        ou_ref[...] = 0.25 * (su_ref[...] + sc[:, 4:5] * pu_ref[...])
        oi_ref[...] = 0.25 * (si_ref[...] + sc[:, 5:6] * pi_ref[...])

    return pl.pallas_call(
        body,
        grid=(_GRID,),
        in_specs=[_tab_spec, _tab_spec, _scT_spec, _tab_spec, _tab_spec],
        out_specs=(_tab_spec,) * 2,
        out_shape=(jax.ShapeDtypeStruct((NPAD, D), _f32),) * 2,
    )(pu, pi, scT, su, si)


def kernel(h_user, h_item, edge_index0, edge_index1, edge_index2):
    hu = jnp.pad(h_user, ((0, NPAD - N), (0, 0)))
    hi = jnp.pad(h_item, ((0, NPAD - N), (0, 0)))

    def prep(e):
        return jnp.pad(e, ((0, 0), (0, EPAD - E)), constant_values=PADROW)

    epad = [prep(edge_index0), prep(edge_index1), prep(edge_index2)]
    edges = [e.reshape(2, ECH, KR) for e in epad]

    deg = _deg_kernel(*epad)
    scT, gu, gi = _head(deg.T, hu, hi)
    su, si = hu, hi
    for l in range(3):
        pu, pi = _edge_kernel(gu, gi, edges[l])
        if l < 2:
            su, si, gu, gi = _combine(pu, pi, scT, su, si, l)
        else:
            ou, oi = _final(pu, pi, scT, su, si)
    return (ou[:N], oi[:N])


# speedup vs baseline: 1.0144x; 1.0144x over previous
import functools

import jax
import jax.numpy as jnp
from jax import lax
from jax.experimental import pallas as pl
from jax.experimental.pallas import tpu as pltpu
from jax.experimental.pallas import tpu_sc as plsc

N = 5000
D = 128
E = 320000
NC = 2
NS = 16
NPAD = 5120
NPT = NPAD // NS
K = 128
NCHUNK = 2560
EPAD = NCHUNK * K
CPT = NCHUNK // NS
HCPT = CPT // 2
KR = 64
ECH = EPAD // KR
CPT2 = ECH // NS
QC = CPT2 // 4
SQ = 56
HQ = QC - SQ
PADROW = NPAD - 1
EPK = EPAD // NS

_mesh = plsc.VectorSubcoreMesh(
    core_axis_name="c", subcore_axis_name="s", num_cores=NC, num_subcores=NS)

_sc_params = pltpu.CompilerParams(use_tc_tiling_on_sc=False)

_f32 = jnp.float32
_i32 = jnp.int32


@functools.partial(
    pl.kernel,
    out_type=jax.ShapeDtypeStruct((6, NPAD), _f32),
    mesh=_mesh,
    compiler_params=_sc_params,
    scratch_types=[
        pltpu.VMEM((2, EPK), _i32),
        pltpu.VMEM((EPK,), _f32),
        pltpu.VMEM((NPT,), _f32),
        [pltpu.VMEM_SHARED((NPAD,), _f32) for _ in range(3)],
        [pltpu.SemaphoreType.DMA for _ in range(2)],
    ],
)
def _deg_kernel(e0, e1, e2, out, idx_v, ones_v, buf_v, hists, dsems):
    c = lax.axis_index("c")
    s = lax.axis_index("s")

    @pl.loop(0, EPK // 16)
    def _(q):
        ones_v[pl.ds(q * 16, 16)] = jnp.ones((16,), _f32)

    @pl.loop(0, NPT // 16)
    def _(q):
        buf_v[pl.ds(q * 16, 16)] = jnp.zeros((16,), _f32)

    for h in hists:
        pltpu.sync_copy(buf_v, h.at[pl.ds(s * NPT, NPT)])
    plsc.subcore_barrier()

    es = (e0, e1, e2)
    for li in range(3):
        b = li % 2
        if li >= 2:
            pltpu.make_async_copy(ones_v, hists[li - 2].at[idx_v.at[b]],
                                  dsems[b]).wait()
        pltpu.sync_copy(es[li].at[c, pl.ds(s * EPK, EPK)], idx_v.at[b])
        pltpu.async_copy(ones_v, hists[li].at[idx_v.at[b]], dsems[b],
                         add=True)
    for li in (1, 2):
        pltpu.make_async_copy(ones_v, hists[li].at[idx_v.at[li % 2]],
                              dsems[li % 2]).wait()

    plsc.subcore_barrier()
    for li in range(3):
        pltpu.sync_copy(hists[li].at[pl.ds(s * NPT, NPT)], buf_v)
        pltpu.sync_copy(buf_v, out.at[2 * li + c, pl.ds(s * NPT, NPT)])


@functools.partial(
    pl.kernel,
    out_type=(jax.ShapeDtypeStruct((NPAD, D), _f32),
              jax.ShapeDtypeStruct((NPAD, D), _f32)),
    mesh=_mesh,
    compiler_params=_sc_params,
    scratch_types=[
        pltpu.VMEM((QC, KR), _i32),
        pltpu.VMEM((QC, KR), _i32),
        pltpu.VMEM((4, KR, D), _f32),
        pltpu.VMEM_SHARED((NPAD, D), _f32),
        pltpu.VMEM_SHARED((NPAD, D), _f32),
        [pltpu.SemaphoreType.DMA for _ in range(4)],
        [pltpu.SemaphoreType.DMA for _ in range(4)],
    ],
)
def _edge_kernel(gu, gi, e, pu, pi, gv, sv, rb, tab, acc, gsems, ssems):
    c = lax.axis_index("c")
    s = lax.axis_index("s")

    @pl.when(c == 0)
    def _():
        pltpu.async_copy(gi.at[pl.ds(s * NPT, NPT)],
                         tab.at[pl.ds(s * NPT, NPT)], gsems[2])

    @pl.when(c == 1)
    def _():
        pltpu.async_copy(gu.at[pl.ds(s * NPT, NPT)],
                         tab.at[pl.ds(s * NPT, NPT)], gsems[2])

    @pl.loop(0, KR)
    def _(i):
        @pl.loop(0, D // 16)
        def _(q):
            rb[0, i, pl.ds(q * 16, 16)] = jnp.zeros((16,), _f32)

    for r in range(NPT // KR):
        pltpu.sync_copy(rb.at[0], acc.at[pl.ds(s * NPT + r * KR, KR)])
    pltpu.sync_copy(e.at[1 - c, pl.ds(s * CPT2, QC)], gv)
    pltpu.sync_copy(e.at[c, pl.ds(s * CPT2, QC)], sv)

    @pl.when(c == 0)
    def _():
        pltpu.make_async_copy(gi.at[pl.ds(s * NPT, NPT)],
                              tab.at[pl.ds(s * NPT, NPT)], gsems[2]).wait()

    @pl.when(c == 1)
    def _():
        pltpu.make_async_copy(gu.at[pl.ds(s * NPT, NPT)],
                              tab.at[pl.ds(s * NPT, NPT)], gsems[2]).wait()
    plsc.subcore_barrier()

    for quarter in range(4):
        base = s * CPT2 + quarter * QC
        if quarter > 0:
            pltpu.sync_copy(e.at[1 - c, pl.ds(base, QC)], gv)
            pltpu.sync_copy(e.at[c, pl.ds(base, QC)], sv)
        for b in range(2):
            pltpu.async_copy(tab.at[gv.at[b]], rb.at[b], gsems[b])

        @pl.loop(0, QC, step=4)
        def _(j):
            for b in range(4):
                jj = j + b
                b2 = (b + 2) % 4
                pltpu.make_async_copy(tab.at[gv.at[jj]], rb.at[b],
                                      gsems[b]).wait()
                pltpu.async_copy(rb.at[b], acc.at[sv.at[jj]], ssems[b],
                                 add=True)

                @pl.when(jj >= 2)
                def _():
                    pltpu.make_async_copy(rb.at[b2], acc.at[sv.at[jj - 2]],
                                          ssems[b2]).wait()

                @pl.when(jj + 2 < QC)
                def _():
                    pltpu.async_copy(tab.at[gv.at[jj + 2]], rb.at[b2],
                                     gsems[b2])

        pltpu.make_async_copy(rb.at[2], acc.at[sv.at[QC - 2]],
                              ssems[2]).wait()
        pltpu.make_async_copy(rb.at[3], acc.at[sv.at[QC - 1]],
                              ssems[3]).wait()

    plsc.subcore_barrier()

    def readout(out):
        nr = NPT // KR
        for r in range(nr):
            b = r % 2
            if r >= 2:
                pltpu.make_async_copy(
                    rb.at[b], out.at[pl.ds(s * NPT + (r - 2) * KR, KR)],
                    gsems[b]).wait()
            pltpu.sync_copy(acc.at[pl.ds(s * NPT + r * KR, KR)], rb.at[b])
            pltpu.async_copy(rb.at[b], out.at[pl.ds(s * NPT + r * KR, KR)],
                             gsems[b])
        for r in (nr - 2, nr - 1):
            pltpu.make_async_copy(
                rb.at[r % 2], out.at[pl.ds(s * NPT + r * KR, KR)],
                gsems[r % 2]).wait()

    @pl.when(c == 0)
    def _():
        readout(pu)

    @pl.when(c == 1)
    def _():
        readout(pi)


_GRID = 8
_RB = NPAD // _GRID

_tab_spec = pl.BlockSpec((_RB, D), lambda i: (i, 0))
_scT_spec = pl.BlockSpec((_RB, 6), lambda i: (i, 0))


def _head(degT, hu, hi):
    def body(degT_ref, hu_ref, hi_ref, scT_ref, gu_ref, gi_ref):
        scT = lax.rsqrt(jnp.maximum(degT_ref[...], 1.0))
        scT_ref[...] = scT
        gu_ref[...] = hu_ref[...] * scT[:, 0:1]
        gi_ref[...] = hi_ref[...] * scT[:, 1:2]

    return pl.pallas_call(
        body,
        grid=(_GRID,),
        in_specs=[_scT_spec, _tab_spec, _tab_spec],
        out_specs=(_scT_spec, _tab_spec, _tab_spec),
        out_shape=(jax.ShapeDtypeStruct((NPAD, 6), _f32),
                   jax.ShapeDtypeStruct((NPAD, D), _f32),
                   jax.ShapeDtypeStruct((NPAD, D), _f32)),
    )(degT, hu, hi)


def _combine(pu, pi, scT, su, si, l):
    def body(pu_ref, pi_ref, scT_ref, su_ref, si_ref,
             nsu_ref, nsi_ref, gu_ref, gi_ref):
        sc = scT_ref[...]
        hu1 = sc[:, 2 * l:2 * l + 1] * pu_ref[...]
        hi1 = sc[:, 2 * l + 1:2 * l + 2] * pi_ref[...]
        nsu_ref[...] = su_ref[...] + hu1
        nsi_ref[...] = si_ref[...] + hi1
        gu_ref[...] = sc[:, 2 * l + 2:2 * l + 3] * hu1
        gi_ref[...] = sc[:, 2 * l + 3:2 * l + 4] * hi1

    return pl.pallas_call(
        body,
        grid=(_GRID,),
        in_specs=[_tab_spec, _tab_spec, _scT_spec, _tab_spec, _tab_spec],
        out_specs=(_tab_spec,) * 4,
        out_shape=(jax.ShapeDtypeStruct((NPAD, D), _f32),) * 4,
    )(pu, pi, scT, su, si)


def _final(pu, pi, scT, su, si):
    def body(pu_ref, pi_ref, scT_ref, su_ref, si_ref, ou_ref, oi_ref):
        sc = scT_ref[...]
        ou_ref[...] = 0.25 * (su_ref[...] + sc[:, 4:5] * pu_ref[...])
        oi_ref[...] = 0.25 * (si_ref[...] + sc[:, 5:6] * pi_ref[...])

    return pl.pallas_call(
        body,
        grid=(_GRID,),
        in_specs=[_tab_spec, _tab_spec, _scT_spec, _tab_spec, _tab_spec],
        out_specs=(_tab_spec,) * 2,
        out_shape=(jax.ShapeDtypeStruct((NPAD, D), _f32),) * 2,
    )(pu, pi, scT, su, si)


def kernel(h_user, h_item, edge_index0, edge_index1, edge_index2):
    hu = jnp.pad(h_user, ((0, NPAD - N), (0, 0)))
    hi = jnp.pad(h_item, ((0, NPAD - N), (0, 0)))

    def prep(e):
        return jnp.pad(e, ((0, 0), (0, EPAD - E)), constant_values=PADROW)

    epad = [prep(edge_index0), prep(edge_index1), prep(edge_index2)]
    edges = [e.reshape(2, ECH, KR) for e in epad]

    deg = _deg_kernel(*epad)
    scT, gu, gi = _head(deg.T, hu, hi)
    su, si = hu, hi
    for l in range(3):
        pu, pi = _edge_kernel(gu, gi, edges[l])
        if l < 2:
            su, si, gu, gi = _combine(pu, pi, scT, su, si, l)
        else:
            ou, oi = _final(pu, pi, scT, su, si)
    return (ou[:N], oi[:N])
